# baseline (speedup 1.0000x reference)
import functools

import jax
import jax.numpy as jnp
from jax import lax
from jax.experimental import pallas as pl
from jax.experimental.pallas import tpu as pltpu
from jax.experimental.pallas import tpu_sc as plsc

E = 64
TOP_K = 2
D = 768
H = 2048
T = 2048
AUX_COEFF = 0.01

BLK = 128
MAX_B = (2 * T) // BLK + E
P = MAX_B * BLK

NC, NS = 2, 16
NW = NC * NS

def _sc_mesh():
    return plsc.VectorSubcoreMesh(core_axis_name="c", subcore_axis_name="s")


def _router_kernel(x_ref, wr_ref, topi_ref, w_ref, aux_ref):
    xx = x_ref[...]
    wr = wr_ref[...]
    logits = lax.dot_general(xx, wr, (((1,), (1,)), ((), ())),
                             preferred_element_type=jnp.float32)
    ids = lax.broadcasted_iota(jnp.int32, (T, E), 1)
    m1 = jnp.max(logits, axis=1, keepdims=True)
    i1 = jnp.min(jnp.where(logits == m1, ids, E), axis=1, keepdims=True)
    masked = jnp.where(ids == i1, -jnp.inf, logits)
    m2 = jnp.max(masked, axis=1, keepdims=True)
    i2 = jnp.min(jnp.where(masked == m2, ids, E), axis=1, keepdims=True)
    w0 = 1.0 / (1.0 + jnp.exp(m2 - m1))
    ez = jnp.exp(logits - m1)
    gates = ez / jnp.sum(ez, axis=1, keepdims=True)
    pm = jnp.mean(gates, axis=0, keepdims=True)
    fm = jnp.mean((ids == i1).astype(jnp.float32), axis=0, keepdims=True)
    aux_ref[...] = AUX_COEFF * E * jnp.sum(fm * pm, keepdims=True)
    topi_ref[...] = jnp.concatenate([i1, i2], axis=1)
    w_ref[...] = jnp.concatenate([w0, 1.0 - w0], axis=1)


def _run_router(xf, Wr):
    return pl.pallas_call(
        _router_kernel,
        out_shape=(
            jax.ShapeDtypeStruct((T, TOP_K), jnp.int32),
            jax.ShapeDtypeStruct((T, TOP_K), jnp.float32),
            jax.ShapeDtypeStruct((1, 1), jnp.float32),
        ),
    )(xf, Wr)


def _ffn_body(xb, wg, wu, wd):
    hg = lax.dot_general(xb, wg, (((1,), (1,)), ((), ())),
                         preferred_element_type=jnp.float32)
    hu = lax.dot_general(xb, wu, (((1,), (1,)), ((), ())),
                         preferred_element_type=jnp.float32)
    h = (hg * 0.5 * (1.0 + lax.erf(hg * 0.7071067811865476))) * hu
    return lax.dot_general(h.astype(jnp.bfloat16), wd, (((1,), (1,)), ((), ())),
                           preferred_element_type=jnp.float32)


def _moe_ffn_kernel(be_ref, stok_ref, x_ref, wg_ref, wu_ref, wd_ref, ws_ref,
                    y_ref, xblk_ref):
    del be_ref
    b = pl.program_id(0)
    xb = x_ref[...].astype(jnp.bfloat16)
    y = _ffn_body(xb, wg_ref[0].astype(jnp.bfloat16),
                  wu_ref[0].astype(jnp.bfloat16),
                  wd_ref[0].astype(jnp.bfloat16))
    y_ref[...] = y * ws_ref[...]


def _run_moe_ffn(block_expert, src_tok, xf, Wgate, Wup, Wdown, w_pad):
    grid_spec = pltpu.PrefetchScalarGridSpec(
        num_scalar_prefetch=2,
        grid=(MAX_B,),
        in_specs=[
            pl.BlockSpec((BLK, D), lambda b, be, st: (0, 0)),
            pl.BlockSpec((1, H, D), lambda b, be, st: (be[b], 0, 0)),
            pl.BlockSpec((1, H, D), lambda b, be, st: (be[b], 0, 0)),
            pl.BlockSpec((1, D, H), lambda b, be, st: (be[b], 0, 0)),
            pl.BlockSpec((BLK, 1), lambda b, be, st: (b, 0)),
        ],
        out_specs=pl.BlockSpec((BLK, D), lambda b, be, st: (b, 0)),
        scratch_shapes=[pltpu.VMEM((BLK, D), jnp.float32)],
    )
    return pl.pallas_call(
        _moe_ffn_kernel,
        grid_spec=grid_spec,
        out_shape=jax.ShapeDtypeStruct((P, D), jnp.float32),
    )(block_expert, src_tok, xf, Wgate, Wup, Wdown, w_pad)


def _shared_ffn_kernel(x_ref, wg_ref, wu_ref, wd_ref, y_ref):
    xb = x_ref[...].astype(jnp.bfloat16)
    y_ref[...] = _ffn_body(xb, wg_ref[...].astype(jnp.bfloat16),
                           wu_ref[...].astype(jnp.bfloat16),
                           wd_ref[...].astype(jnp.bfloat16))


def _run_shared_ffn(xf, Ws_gate, Ws_up, Ws_down):
    sblk = 256
    return pl.pallas_call(
        _shared_ffn_kernel,
        grid=(T // sblk,),
        in_specs=[
            pl.BlockSpec((sblk, D), lambda i: (i, 0)),
            pl.BlockSpec((H, D), lambda i: (0, 0)),
            pl.BlockSpec((H, D), lambda i: (0, 0)),
            pl.BlockSpec((D, H), lambda i: (0, 0)),
        ],
        out_specs=pl.BlockSpec((sblk, D), lambda i: (i, 0)),
        out_shape=jax.ShapeDtypeStruct((T, D), jnp.float32),
    )(xf, Ws_gate, Ws_up, Ws_down)


def _sc_gather(xf, src_tok):
    dd = D
    rows_w = P // NW
    ch = 32
    nch = rows_w // ch

    @functools.partial(
        pl.kernel, mesh=_sc_mesh(),
        out_type=jax.ShapeDtypeStruct((P, dd), jnp.float32),
        scratch_types=[
            pltpu.VMEM((rows_w,), jnp.int32),
            pltpu.VMEM((ch, dd), jnp.float32),
            pltpu.VMEM((ch, dd), jnp.float32),
            pltpu.SemaphoreType.DMA,
            pltpu.SemaphoreType.DMA,
            pltpu.SemaphoreType.DMA,
        ],
    )
    def gather_kernel(x_hbm, idx_hbm, out_hbm, idx_v, r0, r1, sg, sw0, sw1):
        wid = lax.axis_index("s") * NC + lax.axis_index("c")
        base = wid * rows_w
        pltpu.sync_copy(idx_hbm.at[pl.ds(base, rows_w)], idx_v)
        bufs, wsems = (r0, r1), (sw0, sw1)
        writes = [None, None]
        for j in range(nch):
            buf, wsem = bufs[j % 2], wsems[j % 2]
            if writes[j % 2] is not None:
                writes[j % 2].wait()
            pltpu.async_copy(
                x_hbm.at[idx_v.at[pl.ds(j * ch, ch)]], buf, sg).wait()
            writes[j % 2] = pltpu.async_copy(
                buf, out_hbm.at[pl.ds(base + j * ch, ch)], wsem)
        for wr in writes:
            if wr is not None:
                wr.wait()

    return gather_kernel(xf, src_tok)


def _sc_combine(y_padded, pos0, pos1, shared):
    tok_w = T // NW
    ch = 32

    @functools.partial(
        pl.kernel, mesh=_sc_mesh(),
        out_type=jax.ShapeDtypeStruct((T, D), jnp.float32),
        scratch_types=[
            pltpu.VMEM((ch,), jnp.int32),
            pltpu.VMEM((ch, D), jnp.float32),
            pltpu.VMEM((ch, D), jnp.float32),
            pltpu.VMEM((ch, D), jnp.float32),
            pltpu.SemaphoreType.DMA,
        ],
    )
    def combine_kernel(y_hbm, p0_hbm, p1_hbm, sh_hbm, out_hbm,
                       idx_v, a_v, b_v, s_v, sem):
        wid = lax.axis_index("s") * NC + lax.axis_index("c")

        @pl.loop(0, tok_w // ch)
        def _(j):
            base = wid * tok_w + j * ch
            pltpu.sync_copy(p0_hbm.at[pl.ds(base, ch)], idx_v)
            pltpu.async_copy(y_hbm.at[idx_v], a_v, sem).wait()
            pltpu.sync_copy(p1_hbm.at[pl.ds(base, ch)], idx_v)
            pltpu.async_copy(y_hbm.at[idx_v], b_v, sem).wait()
            pltpu.sync_copy(sh_hbm.at[pl.ds(base, ch)], s_v)

            @pl.loop(0, ch)
            def _(r):
                @pl.loop(0, D, step=16)
                def _(cc):
                    slc = (pl.ds(r, 1), pl.ds(cc, 16))
                    a_v.at[*slc][...] = (a_v.at[*slc][...] +
                                         b_v.at[*slc][...] +
                                         s_v.at[*slc][...])

            pltpu.sync_copy(a_v, out_hbm.at[pl.ds(base, ch)])

    return combine_kernel(y_padded, pos0, pos1, shared)


def kernel(x, Wr, Wgate, Wup, Wdown, Ws_gate, Ws_up, Ws_down):
    xf = x.reshape(T, D)

    top_i, w, aux = _run_router(xf, Wr)

    keys = jnp.concatenate([top_i[:, 0], top_i[:, 1]])
    toks = jnp.concatenate([jnp.arange(T, dtype=jnp.int32)] * 2)
    oh = (keys[:, None] == jnp.arange(E, dtype=jnp.int32)[None, :])
    ohi = oh.astype(jnp.int32)
    counts = jnp.sum(ohi, axis=0)
    rank = jnp.sum((jnp.cumsum(ohi, axis=0) - ohi) * ohi, axis=1)
    nblk = (counts + BLK - 1) // BLK
    boff = jnp.concatenate([jnp.zeros((1,), jnp.int32),
                            jnp.cumsum(nblk)[:-1].astype(jnp.int32)])
    block_expert = jnp.clip(
        jnp.repeat(jnp.arange(E, dtype=jnp.int32), nblk,
                   total_repeat_length=MAX_B), 0, E - 1)
    dest = boff[keys] * BLK + rank
    src_tok = jnp.zeros((P,), jnp.int32).at[dest].set(
        toks, unique_indices=True)
    wflat = jnp.concatenate([w[:, 0], w[:, 1]])
    w_pad = jnp.zeros((P, 1), jnp.float32).at[dest, 0].set(
        wflat, unique_indices=True)
    pos0 = dest[:T]
    pos1 = dest[T:]

    block_expert = jnp.arange(MAX_B, dtype=jnp.int32) % E
    src_tok = jnp.arange(P, dtype=jnp.int32) % T
    w_pad = jnp.ones((P, 1), jnp.float32)
    pos0 = jnp.arange(T, dtype=jnp.int32)
    pos1 = jnp.arange(T, dtype=jnp.int32) + T
    shared = _run_shared_ffn(xf, Ws_gate, Ws_up, Ws_down)
    y_padded = _run_moe_ffn(block_expert, src_tok, xf, Wgate, Wup, Wdown,
                            w_pad)
    out = _sc_combine(y_padded, pos0, pos1, shared)

    return out.reshape(x.shape), aux.reshape(())


# speedup vs baseline: 1.0017x; 1.0017x over previous
import functools

import jax
import jax.numpy as jnp
from jax import lax
from jax.experimental import pallas as pl
from jax.experimental.pallas import tpu as pltpu
from jax.experimental.pallas import tpu_sc as plsc

E = 64
TOP_K = 2
D = 768
H = 2048
T = 2048
AUX_COEFF = 0.01

BLK = 256
MAX_B = (2 * T) // BLK + E
P = MAX_B * BLK

NC, NS = 2, 16
NW = NC * NS

def _sc_mesh():
    return plsc.VectorSubcoreMesh(core_axis_name="c", subcore_axis_name="s")


def _router_kernel(x_ref, wr_ref, topi_ref, w_ref, aux_ref):
    xx = x_ref[...]
    wr = wr_ref[...]
    logits = lax.dot_general(xx, wr, (((1,), (1,)), ((), ())),
                             preferred_element_type=jnp.float32)
    ids = lax.broadcasted_iota(jnp.int32, (T, E), 1)
    m1 = jnp.max(logits, axis=1, keepdims=True)
    i1 = jnp.min(jnp.where(logits == m1, ids, E), axis=1, keepdims=True)
    masked = jnp.where(ids == i1, -jnp.inf, logits)
    m2 = jnp.max(masked, axis=1, keepdims=True)
    i2 = jnp.min(jnp.where(masked == m2, ids, E), axis=1, keepdims=True)
    w0 = 1.0 / (1.0 + jnp.exp(m2 - m1))
    ez = jnp.exp(logits - m1)
    gates = ez / jnp.sum(ez, axis=1, keepdims=True)
    pm = jnp.mean(gates, axis=0, keepdims=True)
    fm = jnp.mean((ids == i1).astype(jnp.float32), axis=0, keepdims=True)
    aux_ref[...] = AUX_COEFF * E * jnp.sum(fm * pm, keepdims=True)
    topi_ref[...] = jnp.concatenate([i1, i2], axis=1)
    w_ref[...] = jnp.concatenate([w0, 1.0 - w0], axis=1)


def _run_router(xf, Wr):
    return pl.pallas_call(
        _router_kernel,
        out_shape=(
            jax.ShapeDtypeStruct((T, TOP_K), jnp.int32),
            jax.ShapeDtypeStruct((T, TOP_K), jnp.float32),
            jax.ShapeDtypeStruct((1, 1), jnp.float32),
        ),
    )(xf, Wr)


def _ffn_body(xb, wg, wu, wd):
    hg = lax.dot_general(xb, wg, (((1,), (1,)), ((), ())),
                         preferred_element_type=jnp.float32)
    hu = lax.dot_general(xb, wu, (((1,), (1,)), ((), ())),
                         preferred_element_type=jnp.float32)
    h = (hg * 0.5 * (1.0 + lax.erf(hg * 0.7071067811865476))) * hu
    return lax.dot_general(h.astype(jnp.bfloat16), wd, (((1,), (1,)), ((), ())),
                           preferred_element_type=jnp.float32)


def _moe_ffn_kernel(be_ref, stok_ref, x_ref, wg_ref, wu_ref, wd_ref, ws_ref,
                    y_ref, xblk_ref):
    del be_ref
    b = pl.program_id(0)
    for r in range(BLK):
        i = stok_ref[b * BLK + r]
        xblk_ref[pl.ds(r, 1), :] = x_ref[pl.ds(i, 1), :]
    xb = xblk_ref[...].astype(jnp.bfloat16)
    y = _ffn_body(xb, wg_ref[0].astype(jnp.bfloat16),
                  wu_ref[0].astype(jnp.bfloat16),
                  wd_ref[0].astype(jnp.bfloat16))
    y_ref[...] = y * ws_ref[...]


def _run_moe_ffn(block_expert, src_tok, xf, Wgate, Wup, Wdown, w_pad):
    grid_spec = pltpu.PrefetchScalarGridSpec(
        num_scalar_prefetch=2,
        grid=(MAX_B,),
        in_specs=[
            pl.BlockSpec((T, D), lambda b, be, st: (0, 0)),
            pl.BlockSpec((1, H, D), lambda b, be, st: (be[b], 0, 0)),
            pl.BlockSpec((1, H, D), lambda b, be, st: (be[b], 0, 0)),
            pl.BlockSpec((1, D, H), lambda b, be, st: (be[b], 0, 0)),
            pl.BlockSpec((BLK, 1), lambda b, be, st: (b, 0)),
        ],
        out_specs=pl.BlockSpec((BLK, D), lambda b, be, st: (b, 0)),
        scratch_shapes=[pltpu.VMEM((BLK, D), jnp.float32)],
    )
    return pl.pallas_call(
        _moe_ffn_kernel,
        grid_spec=grid_spec,
        out_shape=jax.ShapeDtypeStruct((P, D), jnp.float32),
    )(block_expert, src_tok, xf, Wgate, Wup, Wdown, w_pad)


def _shared_ffn_kernel(x_ref, wg_ref, wu_ref, wd_ref, y_ref):
    xb = x_ref[...].astype(jnp.bfloat16)
    y_ref[...] = _ffn_body(xb, wg_ref[...].astype(jnp.bfloat16),
                           wu_ref[...].astype(jnp.bfloat16),
                           wd_ref[...].astype(jnp.bfloat16))


def _run_shared_ffn(xf, Ws_gate, Ws_up, Ws_down):
    sblk = 256
    return pl.pallas_call(
        _shared_ffn_kernel,
        grid=(T // sblk,),
        in_specs=[
            pl.BlockSpec((sblk, D), lambda i: (i, 0)),
            pl.BlockSpec((H, D), lambda i: (0, 0)),
            pl.BlockSpec((H, D), lambda i: (0, 0)),
            pl.BlockSpec((D, H), lambda i: (0, 0)),
        ],
        out_specs=pl.BlockSpec((sblk, D), lambda i: (i, 0)),
        out_shape=jax.ShapeDtypeStruct((T, D), jnp.float32),
    )(xf, Ws_gate, Ws_up, Ws_down)


def _sc_gather(xf, src_tok):
    dd = D
    rows_w = P // NW
    ch = 32
    nch = rows_w // ch

    @functools.partial(
        pl.kernel, mesh=_sc_mesh(),
        out_type=jax.ShapeDtypeStruct((P, dd), jnp.float32),
        scratch_types=[
            pltpu.VMEM((rows_w,), jnp.int32),
            pltpu.VMEM((ch, dd), jnp.float32),
            pltpu.VMEM((ch, dd), jnp.float32),
            pltpu.SemaphoreType.DMA,
            pltpu.SemaphoreType.DMA,
            pltpu.SemaphoreType.DMA,
        ],
    )
    def gather_kernel(x_hbm, idx_hbm, out_hbm, idx_v, r0, r1, sg, sw0, sw1):
        wid = lax.axis_index("s") * NC + lax.axis_index("c")
        base = wid * rows_w
        pltpu.sync_copy(idx_hbm.at[pl.ds(base, rows_w)], idx_v)
        bufs, wsems = (r0, r1), (sw0, sw1)
        writes = [None, None]
        for j in range(nch):
            buf, wsem = bufs[j % 2], wsems[j % 2]
            if writes[j % 2] is not None:
                writes[j % 2].wait()
            pltpu.async_copy(
                x_hbm.at[idx_v.at[pl.ds(j * ch, ch)]], buf, sg).wait()
            writes[j % 2] = pltpu.async_copy(
                buf, out_hbm.at[pl.ds(base + j * ch, ch)], wsem)
        for wr in writes:
            if wr is not None:
                wr.wait()

    return gather_kernel(xf, src_tok)


def _sc_combine(y_padded, pos0, pos1, shared):
    tok_w = T // NW
    ch = 32

    @functools.partial(
        pl.kernel, mesh=_sc_mesh(),
        out_type=jax.ShapeDtypeStruct((T, D), jnp.float32),
        scratch_types=[
            pltpu.VMEM((ch,), jnp.int32),
            pltpu.VMEM((ch, D), jnp.float32),
            pltpu.VMEM((ch, D), jnp.float32),
            pltpu.VMEM((ch, D), jnp.float32),
            pltpu.SemaphoreType.DMA,
        ],
    )
    def combine_kernel(y_hbm, p0_hbm, p1_hbm, sh_hbm, out_hbm,
                       idx_v, a_v, b_v, s_v, sem):
        wid = lax.axis_index("s") * NC + lax.axis_index("c")

        @pl.loop(0, tok_w // ch)
        def _(j):
            base = wid * tok_w + j * ch
            pltpu.sync_copy(p0_hbm.at[pl.ds(base, ch)], idx_v)
            pltpu.async_copy(y_hbm.at[idx_v], a_v, sem).wait()
            pltpu.sync_copy(p1_hbm.at[pl.ds(base, ch)], idx_v)
            pltpu.async_copy(y_hbm.at[idx_v], b_v, sem).wait()
            pltpu.sync_copy(sh_hbm.at[pl.ds(base, ch)], s_v)

            @pl.loop(0, ch)
            def _(r):
                @pl.loop(0, D, step=16)
                def _(cc):
                    slc = (pl.ds(r, 1), pl.ds(cc, 16))
                    a_v.at[*slc][...] = (a_v.at[*slc][...] +
                                         b_v.at[*slc][...] +
                                         s_v.at[*slc][...])

            pltpu.sync_copy(a_v, out_hbm.at[pl.ds(base, ch)])

    return combine_kernel(y_padded, pos0, pos1, shared)


def kernel(x, Wr, Wgate, Wup, Wdown, Ws_gate, Ws_up, Ws_down):
    xf = x.reshape(T, D)

    top_i, w, aux = _run_router(xf, Wr)

    keys = jnp.concatenate([top_i[:, 0], top_i[:, 1]])
    toks = jnp.concatenate([jnp.arange(T, dtype=jnp.int32)] * 2)
    oh = (keys[:, None] == jnp.arange(E, dtype=jnp.int32)[None, :])
    ohi = oh.astype(jnp.int32)
    counts = jnp.sum(ohi, axis=0)
    rank = jnp.sum((jnp.cumsum(ohi, axis=0) - ohi) * ohi, axis=1)
    nblk = (counts + BLK - 1) // BLK
    boff = jnp.concatenate([jnp.zeros((1,), jnp.int32),
                            jnp.cumsum(nblk)[:-1].astype(jnp.int32)])
    block_expert = jnp.clip(
        jnp.repeat(jnp.arange(E, dtype=jnp.int32), nblk,
                   total_repeat_length=MAX_B), 0, E - 1)
    dest = boff[keys] * BLK + rank
    src_tok = jnp.zeros((P,), jnp.int32).at[dest].set(
        toks, unique_indices=True)
    wflat = jnp.concatenate([w[:, 0], w[:, 1]])
    w_pad = jnp.zeros((P, 1), jnp.float32).at[dest, 0].set(
        wflat, unique_indices=True)
    pos0 = dest[:T]
    pos1 = dest[T:]

    shared = _run_shared_ffn(xf, Ws_gate, Ws_up, Ws_down)
    y_padded = _run_moe_ffn(block_expert, src_tok, xf, Wgate, Wup, Wdown,
                            w_pad)
    out = _sc_combine(y_padded, pos0, pos1, shared)

    return out.reshape(x.shape), aux.reshape(())


# speedup vs baseline: 1.0721x; 1.0703x over previous
import functools

import jax
import jax.numpy as jnp
from jax import lax
from jax.experimental import pallas as pl
from jax.experimental.pallas import tpu as pltpu
from jax.experimental.pallas import tpu_sc as plsc

E = 64
TOP_K = 2
D = 768
H = 2048
T = 2048
AUX_COEFF = 0.01

BLK = 256
MAX_B = (2 * T) // BLK + E
P = MAX_B * BLK

NC, NS = 2, 16
NW = NC * NS

def _sc_mesh():
    return plsc.VectorSubcoreMesh(core_axis_name="c", subcore_axis_name="s")


def _router_kernel(x_ref, wr_ref, topi_ref, w_ref, aux_ref):
    xx = x_ref[...]
    wr = wr_ref[...]
    logits = lax.dot_general(xx, wr, (((1,), (1,)), ((), ())),
                             preferred_element_type=jnp.float32)
    ids = lax.broadcasted_iota(jnp.int32, (T, E), 1)
    m1 = jnp.max(logits, axis=1, keepdims=True)
    i1 = jnp.min(jnp.where(logits == m1, ids, E), axis=1, keepdims=True)
    masked = jnp.where(ids == i1, -jnp.inf, logits)
    m2 = jnp.max(masked, axis=1, keepdims=True)
    i2 = jnp.min(jnp.where(masked == m2, ids, E), axis=1, keepdims=True)
    w0 = 1.0 / (1.0 + jnp.exp(m2 - m1))
    ez = jnp.exp(logits - m1)
    gates = ez / jnp.sum(ez, axis=1, keepdims=True)
    pm = jnp.mean(gates, axis=0, keepdims=True)
    fm = jnp.mean((ids == i1).astype(jnp.float32), axis=0, keepdims=True)
    aux_ref[...] = AUX_COEFF * E * jnp.sum(fm * pm, keepdims=True)
    topi_ref[...] = jnp.concatenate([i1, i2], axis=1)
    w_ref[...] = jnp.concatenate([w0, 1.0 - w0], axis=1)


def _run_router(xf, Wr):
    return pl.pallas_call(
        _router_kernel,
        out_shape=(
            jax.ShapeDtypeStruct((T, TOP_K), jnp.int32),
            jax.ShapeDtypeStruct((T, TOP_K), jnp.float32),
            jax.ShapeDtypeStruct((1, 1), jnp.float32),
        ),
    )(xf, Wr)


def _ffn_body(xb, wg, wu, wd):
    hg = lax.dot_general(xb, wg, (((1,), (1,)), ((), ())),
                         preferred_element_type=jnp.float32)
    hu = lax.dot_general(xb, wu, (((1,), (1,)), ((), ())),
                         preferred_element_type=jnp.float32)
    h = (hg * 0.5 * (1.0 + lax.erf(hg * 0.7071067811865476))) * hu
    return lax.dot_general(h.astype(jnp.bfloat16), wd, (((1,), (1,)), ((), ())),
                           preferred_element_type=jnp.float32)


def _moe_ffn_kernel(be_ref, stok_ref, nact_ref, x_ref, wg_ref, wu_ref, wd_ref,
                    ws_ref, y_ref, xblk_ref):
    del be_ref
    b = pl.program_id(0)

    @pl.when(b < nact_ref[0])
    def _():
        for r in range(BLK):
            i = stok_ref[b * BLK + r]
            xblk_ref[pl.ds(r, 1), :] = x_ref[pl.ds(i, 1), :]
        xb = xblk_ref[...].astype(jnp.bfloat16)
        y = _ffn_body(xb, wg_ref[0].astype(jnp.bfloat16),
                      wu_ref[0].astype(jnp.bfloat16),
                      wd_ref[0].astype(jnp.bfloat16))
        y_ref[...] = y * ws_ref[...]


def _run_moe_ffn(block_expert, src_tok, n_active, xf, Wgate, Wup, Wdown,
                 w_pad):
    grid_spec = pltpu.PrefetchScalarGridSpec(
        num_scalar_prefetch=3,
        grid=(MAX_B,),
        in_specs=[
            pl.BlockSpec((T, D), lambda b, be, st, na: (0, 0)),
            pl.BlockSpec((1, H, D), lambda b, be, st, na: (be[b], 0, 0)),
            pl.BlockSpec((1, H, D), lambda b, be, st, na: (be[b], 0, 0)),
            pl.BlockSpec((1, D, H), lambda b, be, st, na: (be[b], 0, 0)),
            pl.BlockSpec((BLK, 1), lambda b, be, st, na: (b, 0)),
        ],
        out_specs=pl.BlockSpec((BLK, D), lambda b, be, st, na: (b, 0)),
        scratch_shapes=[pltpu.VMEM((BLK, D), jnp.float32)],
    )
    return pl.pallas_call(
        _moe_ffn_kernel,
        grid_spec=grid_spec,
        out_shape=jax.ShapeDtypeStruct((P, D), jnp.float32),
    )(block_expert, src_tok, n_active, xf, Wgate, Wup, Wdown, w_pad)


def _shared_ffn_kernel(x_ref, wg_ref, wu_ref, wd_ref, y_ref):
    xb = x_ref[...].astype(jnp.bfloat16)
    y_ref[...] = _ffn_body(xb, wg_ref[...].astype(jnp.bfloat16),
                           wu_ref[...].astype(jnp.bfloat16),
                           wd_ref[...].astype(jnp.bfloat16))


def _run_shared_ffn(xf, Ws_gate, Ws_up, Ws_down):
    sblk = 256
    return pl.pallas_call(
        _shared_ffn_kernel,
        grid=(T // sblk,),
        in_specs=[
            pl.BlockSpec((sblk, D), lambda i: (i, 0)),
            pl.BlockSpec((H, D), lambda i: (0, 0)),
            pl.BlockSpec((H, D), lambda i: (0, 0)),
            pl.BlockSpec((D, H), lambda i: (0, 0)),
        ],
        out_specs=pl.BlockSpec((sblk, D), lambda i: (i, 0)),
        out_shape=jax.ShapeDtypeStruct((T, D), jnp.float32),
    )(xf, Ws_gate, Ws_up, Ws_down)


def _sc_combine(y_padded, pos0, pos1, shared):
    tok_w = T // NW
    ch = 32

    @functools.partial(
        pl.kernel, mesh=_sc_mesh(),
        out_type=jax.ShapeDtypeStruct((T, D), jnp.float32),
        scratch_types=[
            pltpu.VMEM((ch,), jnp.int32),
            pltpu.VMEM((ch, D), jnp.float32),
            pltpu.VMEM((ch, D), jnp.float32),
            pltpu.VMEM((ch, D), jnp.float32),
            pltpu.SemaphoreType.DMA,
        ],
    )
    def combine_kernel(y_hbm, p0_hbm, p1_hbm, sh_hbm, out_hbm,
                       idx_v, a_v, b_v, s_v, sem):
        wid = lax.axis_index("s") * NC + lax.axis_index("c")

        @pl.loop(0, tok_w // ch)
        def _(j):
            base = wid * tok_w + j * ch
            pltpu.sync_copy(p0_hbm.at[pl.ds(base, ch)], idx_v)
            pltpu.async_copy(y_hbm.at[idx_v], a_v, sem).wait()
            pltpu.sync_copy(p1_hbm.at[pl.ds(base, ch)], idx_v)
            pltpu.async_copy(y_hbm.at[idx_v], b_v, sem).wait()
            pltpu.sync_copy(sh_hbm.at[pl.ds(base, ch)], s_v)

            @pl.loop(0, ch)
            def _(r):
                @pl.loop(0, D, step=16)
                def _(cc):
                    slc = (pl.ds(r, 1), pl.ds(cc, 16))
                    a_v.at[*slc][...] = (a_v.at[*slc][...] +
                                         b_v.at[*slc][...] +
                                         s_v.at[*slc][...])

            pltpu.sync_copy(a_v, out_hbm.at[pl.ds(base, ch)])

    return combine_kernel(y_padded, pos0, pos1, shared)


def kernel(x, Wr, Wgate, Wup, Wdown, Ws_gate, Ws_up, Ws_down):
    xf = x.reshape(T, D)

    top_i, w, aux = _run_router(xf, Wr)

    keys = jnp.concatenate([top_i[:, 0], top_i[:, 1]])
    toks = jnp.concatenate([jnp.arange(T, dtype=jnp.int32)] * 2)
    oh = (keys[:, None] == jnp.arange(E, dtype=jnp.int32)[None, :])
    ohi = oh.astype(jnp.int32)
    counts = jnp.sum(ohi, axis=0)
    rank = jnp.sum((jnp.cumsum(ohi, axis=0) - ohi) * ohi, axis=1)
    nblk = (counts + BLK - 1) // BLK
    boff = jnp.concatenate([jnp.zeros((1,), jnp.int32),
                            jnp.cumsum(nblk)[:-1].astype(jnp.int32)])
    block_expert = jnp.clip(
        jnp.repeat(jnp.arange(E, dtype=jnp.int32), nblk,
                   total_repeat_length=MAX_B), 0, E - 1)
    dest = boff[keys] * BLK + rank
    src_tok = jnp.zeros((P,), jnp.int32).at[dest].set(
        toks, unique_indices=True)
    wflat = jnp.concatenate([w[:, 0], w[:, 1]])
    w_pad = jnp.zeros((P, 1), jnp.float32).at[dest, 0].set(
        wflat, unique_indices=True)
    pos0 = dest[:T]
    pos1 = dest[T:]

    n_active = jnp.sum(nblk, dtype=jnp.int32).reshape(1)

    shared = _run_shared_ffn(xf, Ws_gate, Ws_up, Ws_down)
    y_padded = _run_moe_ffn(block_expert, src_tok, n_active, xf, Wgate, Wup,
                            Wdown, w_pad)
    out = _sc_combine(y_padded, pos0, pos1, shared)

    return out.reshape(x.shape), aux.reshape(())


# speedup vs baseline: 1.0895x; 1.0162x over previous
import functools

import jax
import jax.numpy as jnp
from jax import lax
from jax.experimental import pallas as pl
from jax.experimental.pallas import tpu as pltpu
from jax.experimental.pallas import tpu_sc as plsc

E = 64
TOP_K = 2
D = 768
H = 2048
T = 2048
AUX_COEFF = 0.01

BLK = 256
MAX_B = (2 * T) // BLK + E
P = MAX_B * BLK

NC, NS = 2, 16
NW = NC * NS

def _sc_mesh():
    return plsc.VectorSubcoreMesh(core_axis_name="c", subcore_axis_name="s")


def _router_kernel(x_ref, wr_ref, topi_ref, w_ref, aux_ref):
    xx = x_ref[...]
    wr = wr_ref[...]
    logits = lax.dot_general(xx, wr, (((1,), (1,)), ((), ())),
                             preferred_element_type=jnp.float32)
    ids = lax.broadcasted_iota(jnp.int32, (T, E), 1)
    m1 = jnp.max(logits, axis=1, keepdims=True)
    i1 = jnp.min(jnp.where(logits == m1, ids, E), axis=1, keepdims=True)
    masked = jnp.where(ids == i1, -jnp.inf, logits)
    m2 = jnp.max(masked, axis=1, keepdims=True)
    i2 = jnp.min(jnp.where(masked == m2, ids, E), axis=1, keepdims=True)
    w0 = 1.0 / (1.0 + jnp.exp(m2 - m1))
    ez = jnp.exp(logits - m1)
    gates = ez / jnp.sum(ez, axis=1, keepdims=True)
    pm = jnp.mean(gates, axis=0, keepdims=True)
    fm = jnp.mean((ids == i1).astype(jnp.float32), axis=0, keepdims=True)
    aux_ref[...] = AUX_COEFF * E * jnp.sum(fm * pm, keepdims=True)
    topi_ref[...] = jnp.concatenate([i1, i2], axis=1)
    w_ref[...] = jnp.concatenate([w0, 1.0 - w0], axis=1)


def _run_router(xf, Wr):
    return pl.pallas_call(
        _router_kernel,
        out_shape=(
            jax.ShapeDtypeStruct((T, TOP_K), jnp.int32),
            jax.ShapeDtypeStruct((T, TOP_K), jnp.float32),
            jax.ShapeDtypeStruct((1, 1), jnp.float32),
        ),
    )(xf, Wr)


def _ffn_body(xb, wg, wu, wd):
    hg = lax.dot_general(xb, wg, (((1,), (1,)), ((), ())),
                         preferred_element_type=jnp.float32)
    hu = lax.dot_general(xb, wu, (((1,), (1,)), ((), ())),
                         preferred_element_type=jnp.float32)
    h = (hg * 0.5 * (1.0 + lax.erf(hg * 0.7071067811865476))) * hu
    return lax.dot_general(h.astype(jnp.bfloat16), wd, (((1,), (1,)), ((), ())),
                           preferred_element_type=jnp.float32)


def _moe_ffn_kernel(be_ref, stok_ref, nact_ref, x_ref, wg_ref, wu_ref, wd_ref,
                    ws_ref, y_ref, xblk_ref):
    del be_ref
    b = pl.program_id(0)

    @pl.when(b < nact_ref[0])
    def _():
        for r in range(BLK):
            i = stok_ref[b * BLK + r]
            xblk_ref[pl.ds(r, 1), :] = x_ref[pl.ds(i, 1), :]
        xb = xblk_ref[...].astype(jnp.bfloat16)
        y = _ffn_body(xb, wg_ref[0].astype(jnp.bfloat16),
                      wu_ref[0].astype(jnp.bfloat16),
                      wd_ref[0].astype(jnp.bfloat16))
        y_ref[...] = y * ws_ref[...]


def _run_moe_ffn(block_expert, src_tok, n_active, xf, Wgate, Wup, Wdown,
                 w_pad):
    grid_spec = pltpu.PrefetchScalarGridSpec(
        num_scalar_prefetch=3,
        grid=(MAX_B,),
        in_specs=[
            pl.BlockSpec((T, D), lambda b, be, st, na: (0, 0)),
            pl.BlockSpec((1, H, D), lambda b, be, st, na: (be[b], 0, 0)),
            pl.BlockSpec((1, H, D), lambda b, be, st, na: (be[b], 0, 0)),
            pl.BlockSpec((1, D, H), lambda b, be, st, na: (be[b], 0, 0)),
            pl.BlockSpec((BLK, 1), lambda b, be, st, na: (b, 0)),
        ],
        out_specs=pl.BlockSpec((BLK, D), lambda b, be, st, na: (b, 0)),
        scratch_shapes=[pltpu.VMEM((BLK, D), jnp.float32)],
    )
    return pl.pallas_call(
        _moe_ffn_kernel,
        grid_spec=grid_spec,
        out_shape=jax.ShapeDtypeStruct((P, D), jnp.float32),
    )(block_expert, src_tok, n_active, xf, Wgate, Wup, Wdown, w_pad)


def _shared_ffn_kernel(x_ref, wg_ref, wu_ref, wd_ref, y_ref):
    xb = x_ref[...].astype(jnp.bfloat16)
    y_ref[...] = _ffn_body(xb, wg_ref[...].astype(jnp.bfloat16),
                           wu_ref[...].astype(jnp.bfloat16),
                           wd_ref[...].astype(jnp.bfloat16))


def _run_shared_ffn(xf, Ws_gate, Ws_up, Ws_down):
    sblk = 256
    return pl.pallas_call(
        _shared_ffn_kernel,
        grid=(T // sblk,),
        in_specs=[
            pl.BlockSpec((sblk, D), lambda i: (i, 0)),
            pl.BlockSpec((H, D), lambda i: (0, 0)),
            pl.BlockSpec((H, D), lambda i: (0, 0)),
            pl.BlockSpec((D, H), lambda i: (0, 0)),
        ],
        out_specs=pl.BlockSpec((sblk, D), lambda i: (i, 0)),
        out_shape=jax.ShapeDtypeStruct((T, D), jnp.float32),
    )(xf, Ws_gate, Ws_up, Ws_down)


def _sc_combine(y_padded, pos0, pos1, shared):
    tok_w = T // NW
    ch = 16
    nch = tok_w // ch

    @functools.partial(
        pl.kernel, mesh=_sc_mesh(),
        out_type=jax.ShapeDtypeStruct((T, D), jnp.float32),
        scratch_types=(
            [pltpu.VMEM((tok_w,), jnp.int32),
             pltpu.VMEM((tok_w,), jnp.int32)] +
            [pltpu.VMEM((ch, D), jnp.float32)] * 6 +
            [pltpu.SemaphoreType.DMA] * 4
        ),
    )
    def combine_kernel(y_hbm, p0_hbm, p1_hbm, sh_hbm, out_hbm,
                       i0_v, i1_v, a0, b0, s0, a1, b1, s1,
                       sem0, sem1, wsem0, wsem1):
        wid = lax.axis_index("s") * NC + lax.axis_index("c")
        base = wid * tok_w
        pltpu.sync_copy(p0_hbm.at[pl.ds(base, tok_w)], i0_v)
        pltpu.sync_copy(p1_hbm.at[pl.ds(base, tok_w)], i1_v)
        gsems = (sem0, sem1)
        wsems = (wsem0, wsem1)

        def start(j, a_v, b_v, s_v, gsem):
            pltpu.async_copy(y_hbm.at[i0_v.at[pl.ds(j * ch, ch)]], a_v, gsem)
            pltpu.async_copy(y_hbm.at[i1_v.at[pl.ds(j * ch, ch)]], b_v, gsem)
            pltpu.async_copy(sh_hbm.at[pl.ds(base + j * ch, ch)], s_v, gsem)

        def drain(a_v, b_v, s_v, gsem):
            pltpu.make_async_copy(y_hbm.at[i0_v.at[pl.ds(0, ch)]], a_v,
                                  gsem).wait()
            pltpu.make_async_copy(y_hbm.at[i1_v.at[pl.ds(0, ch)]], b_v,
                                  gsem).wait()
            pltpu.make_async_copy(sh_hbm.at[pl.ds(base, ch)], s_v,
                                  gsem).wait()

        bufsets = [(a0, b0, s0), (a1, b1, s1)]
        start(0, *bufsets[0], gsems[0])
        writes = [None, None]
        for j in range(nch):
            k = j % 2
            a_v, b_v, s_v = bufsets[k]
            drain(a_v, b_v, s_v, gsems[k])
            if j + 1 < nch:
                kn = (j + 1) % 2
                if writes[kn] is not None:
                    writes[kn].wait()
                    writes[kn] = None
                start(j + 1, *bufsets[kn], gsems[kn])

            @pl.loop(0, ch)
            def _(r):
                @pl.loop(0, D, step=16)
                def _(cc):
                    slc = (pl.ds(r, 1), pl.ds(cc, 16))
                    a_v.at[*slc][...] = (a_v.at[*slc][...] +
                                         b_v.at[*slc][...] +
                                         s_v.at[*slc][...])

            writes[k] = pltpu.async_copy(
                a_v, out_hbm.at[pl.ds(base + j * ch, ch)], wsems[k])
        for wr in writes:
            if wr is not None:
                wr.wait()

    return combine_kernel(y_padded, pos0, pos1, shared)


def kernel(x, Wr, Wgate, Wup, Wdown, Ws_gate, Ws_up, Ws_down):
    xf = x.reshape(T, D)

    top_i, w, aux = _run_router(xf, Wr)

    keys = jnp.concatenate([top_i[:, 0], top_i[:, 1]])
    toks = jnp.concatenate([jnp.arange(T, dtype=jnp.int32)] * 2)
    oh = (keys[:, None] == jnp.arange(E, dtype=jnp.int32)[None, :])
    ohi = oh.astype(jnp.int32)
    counts = jnp.sum(ohi, axis=0)
    rank = jnp.sum((jnp.cumsum(ohi, axis=0) - ohi) * ohi, axis=1)
    nblk = (counts + BLK - 1) // BLK
    boff = jnp.concatenate([jnp.zeros((1,), jnp.int32),
                            jnp.cumsum(nblk)[:-1].astype(jnp.int32)])
    block_expert = jnp.clip(
        jnp.repeat(jnp.arange(E, dtype=jnp.int32), nblk,
                   total_repeat_length=MAX_B), 0, E - 1)
    dest = boff[keys] * BLK + rank
    src_tok = jnp.zeros((P,), jnp.int32).at[dest].set(
        toks, unique_indices=True)
    wflat = jnp.concatenate([w[:, 0], w[:, 1]])
    w_pad = jnp.zeros((P, 1), jnp.float32).at[dest, 0].set(
        wflat, unique_indices=True)
    pos0 = dest[:T]
    pos1 = dest[T:]

    n_active = jnp.sum(nblk, dtype=jnp.int32).reshape(1)

    shared = _run_shared_ffn(xf, Ws_gate, Ws_up, Ws_down)
    y_padded = _run_moe_ffn(block_expert, src_tok, n_active, xf, Wgate, Wup,
                            Wdown, w_pad)
    out = _sc_combine(y_padded, pos0, pos1, shared)

    return out.reshape(x.shape), aux.reshape(())
